# baseline (speedup 1.0000x reference)
import functools

import jax
import jax.numpy as jnp
from jax import lax
from jax.experimental import pallas as pl
from jax.experimental.pallas import tpu as pltpu
from jax.experimental.pallas import tpu_sc as plsc

VOCAB_ROWS = 1000000
EMBED_DIM = 32
BBLK = 128
NUM_WORKERS = 32

REPACK_VBLK = 2048
SEG = 512


def _tc_repack_kernel(tt_ref, out_ref):
    seg_n = REPACK_VBLK // SEG
    y = tt_ref[...].T
    r_i = lax.broadcasted_iota(jnp.int32, (SEG // 4, SEG), 0)
    v_i = lax.broadcasted_iota(jnp.int32, (SEG // 4, SEG), 1)
    for j in range(4):
        sel = (v_i == 4 * r_i + j).astype(jnp.float32)
        for g in range(seg_n):
            seg = y[SEG * g : SEG * (g + 1), :]
            picked = jax.lax.dot_general(
                sel,
                seg,
                (((1,), (0,)), ((), ())),
                preferred_element_type=jnp.float32,
            )
            out_ref[
                pl.ds(SEG // 4 * g, SEG // 4),
                pl.ds(EMBED_DIM * j, EMBED_DIM),
            ] = picked


def _tc_repack(table_t):
    return pl.pallas_call(
        _tc_repack_kernel,
        grid=(pl.cdiv(VOCAB_ROWS, REPACK_VBLK),),
        in_specs=[
            pl.BlockSpec((EMBED_DIM, REPACK_VBLK), lambda g: (0, g)),
        ],
        out_specs=pl.BlockSpec(
            (REPACK_VBLK // 4, 4 * EMBED_DIM), lambda g: (g, 0)
        ),
        out_shape=jax.ShapeDtypeStruct(
            (VOCAB_ROWS // 4, 4 * EMBED_DIM), jnp.float32
        ),
    )(table_t)


def _tc_out_transpose_kernel(x_ref, out_ref):
    out_ref[...] = x_ref[...].T


def _tc_out_transpose(x, seq_len, batch):
    return pl.pallas_call(
        _tc_out_transpose_kernel,
        grid=(seq_len // 4,),
        in_specs=[pl.BlockSpec((batch, 4 * EMBED_DIM), lambda g: (g, 0))],
        out_specs=pl.BlockSpec((4 * EMBED_DIM, batch), lambda g: (g, 0)),
        out_shape=jax.ShapeDtypeStruct(
            (seq_len * EMBED_DIM, batch), jnp.float32
        ),
    )(x)


def _sc_gather(sents_t, table_rm):
    seq_len, batch = sents_t.shape

    mesh = plsc.VectorSubcoreMesh(core_axis_name="c", subcore_axis_name="s")

    @functools.partial(
        pl.kernel,
        mesh=mesh,
        out_type=jax.ShapeDtypeStruct(
            (seq_len * batch // 4, 4 * EMBED_DIM), jnp.float32
        ),
        scratch_types=[
            pltpu.VMEM((seq_len, BBLK), jnp.int32),
            pltpu.VMEM((4, BBLK, EMBED_DIM), jnp.float32),
            pltpu.SemaphoreType.DMA,
        ],
        compiler_params=pltpu.CompilerParams(use_tc_tiling_on_sc=False),
    )
    def k(table_hbm, idx_hbm, out_hbm, idx_v, rows_v, gsem):
        wid = lax.axis_index("s") * 2 + lax.axis_index("c")
        b0 = wid * BBLK

        pltpu.sync_copy(idx_hbm.at[:, pl.ds(b0, BBLK)], idx_v)

        def start_gather(s, b):
            pltpu.async_copy(table_hbm.at[idx_v.at[s]], rows_v.at[b], gsem)

        def wait_gather(b):
            pltpu.make_async_copy(
                table_hbm.at[idx_v.at[0]], rows_v.at[b], gsem
            ).wait()

        for b in range(4):
            start_gather(b, b)

        def body(g, carry):
            for b in range(4):
                s = g * 4 + b
                wait_gather(b)
                pltpu.sync_copy(
                    rows_v.at[b],
                    out_hbm.at[
                        pl.ds(g * batch + b0, BBLK),
                        pl.ds(b * EMBED_DIM, EMBED_DIM),
                    ],
                )

                @pl.when(s + 4 < seq_len)
                def _():
                    start_gather(s + 4, b)

            return carry

        lax.fori_loop(0, seq_len // 4, body, 0)

    return k(table_rm, sents_t)


def kernel(sents, table):
    if sents.ndim < 2:
        sents = sents[None, :]
    batch, seq_len = sents.shape

    table_rm = _tc_repack(table.T).reshape(VOCAB_ROWS, EMBED_DIM)
    x2 = _sc_gather(sents.T.astype(jnp.int32), table_rm)
    out2d = _tc_out_transpose(x2, seq_len, batch)
    return out2d.reshape(seq_len, EMBED_DIM, batch).transpose(2, 0, 1)


# speedup vs baseline: 1.0477x; 1.0477x over previous
import functools

import jax
import jax.numpy as jnp
from jax import lax
from jax.experimental import pallas as pl
from jax.experimental.pallas import tpu as pltpu
from jax.experimental.pallas import tpu_sc as plsc

VOCAB_ROWS = 1000000
EMBED_DIM = 32
BBLK = 128
NUM_WORKERS = 32

REPACK_VBLK = 2048
SEG = 512


def _tc_repack_kernel(sel_ref, tt_ref, out_ref):
    sel = sel_ref[...]
    for g in range(REPACK_VBLK // SEG):
        xg = tt_ref[:, SEG * g : SEG * (g + 1)]
        y = jax.lax.dot_general(
            sel,
            xg,
            (((1,), (1,)), ((), ())),
            preferred_element_type=jnp.float32,
        )
        for j in range(4):
            out_ref[
                pl.ds(SEG // 4 * g, SEG // 4),
                pl.ds(EMBED_DIM * j, EMBED_DIM),
            ] = y[SEG // 4 * j : SEG // 4 * (j + 1), :]


def _tc_repack(table_t):
    p_i = lax.broadcasted_iota(jnp.int32, (SEG, SEG), 0)
    v_i = lax.broadcasted_iota(jnp.int32, (SEG, SEG), 1)
    sel = (v_i == 4 * (p_i % (SEG // 4)) + p_i // (SEG // 4)).astype(
        jnp.float32
    )
    return pl.pallas_call(
        _tc_repack_kernel,
        grid=(pl.cdiv(VOCAB_ROWS, REPACK_VBLK),),
        in_specs=[
            pl.BlockSpec((SEG, SEG), lambda g: (0, 0)),
            pl.BlockSpec((EMBED_DIM, REPACK_VBLK), lambda g: (0, g)),
        ],
        out_specs=pl.BlockSpec(
            (REPACK_VBLK // 4, 4 * EMBED_DIM), lambda g: (g, 0)
        ),
        out_shape=jax.ShapeDtypeStruct(
            (VOCAB_ROWS // 4, 4 * EMBED_DIM), jnp.float32
        ),
    )(sel, table_t)


def _tc_out_transpose_kernel(x_ref, out_ref):
    out_ref[...] = x_ref[...].T


def _tc_out_transpose(x, seq_len, batch):
    return pl.pallas_call(
        _tc_out_transpose_kernel,
        grid=(seq_len // 4,),
        in_specs=[pl.BlockSpec((batch, 4 * EMBED_DIM), lambda g: (g, 0))],
        out_specs=pl.BlockSpec((4 * EMBED_DIM, batch), lambda g: (g, 0)),
        out_shape=jax.ShapeDtypeStruct(
            (seq_len * EMBED_DIM, batch), jnp.float32
        ),
    )(x)


def _sc_gather(sents_t, table_rm):
    seq_len, batch = sents_t.shape

    mesh = plsc.VectorSubcoreMesh(core_axis_name="c", subcore_axis_name="s")

    @functools.partial(
        pl.kernel,
        mesh=mesh,
        out_type=jax.ShapeDtypeStruct(
            (seq_len * batch // 4, 4 * EMBED_DIM), jnp.float32
        ),
        scratch_types=[
            pltpu.VMEM((seq_len, BBLK), jnp.int32),
            pltpu.VMEM((4, BBLK, EMBED_DIM), jnp.float32),
            pltpu.SemaphoreType.DMA,
        ],
        compiler_params=pltpu.CompilerParams(use_tc_tiling_on_sc=False),
    )
    def k(table_hbm, idx_hbm, out_hbm, idx_v, rows_v, gsem):
        wid = lax.axis_index("s") * 2 + lax.axis_index("c")
        b0 = wid * BBLK

        pltpu.sync_copy(idx_hbm.at[:, pl.ds(b0, BBLK)], idx_v)

        def start_gather(s, b):
            pltpu.async_copy(table_hbm.at[idx_v.at[s]], rows_v.at[b], gsem)

        def wait_gather(b):
            pltpu.make_async_copy(
                table_hbm.at[idx_v.at[0]], rows_v.at[b], gsem
            ).wait()

        for b in range(4):
            start_gather(b, b)

        def body(g, carry):
            for b in range(4):
                s = g * 4 + b
                wait_gather(b)
                pltpu.sync_copy(
                    rows_v.at[b],
                    out_hbm.at[
                        pl.ds(g * batch + b0, BBLK),
                        pl.ds(b * EMBED_DIM, EMBED_DIM),
                    ],
                )

                @pl.when(s + 4 < seq_len)
                def _():
                    start_gather(s + 4, b)

            return carry

        lax.fori_loop(0, seq_len // 4, body, 0)

    return k(table_rm, sents_t)


def kernel(sents, table):
    if sents.ndim < 2:
        sents = sents[None, :]
    batch, seq_len = sents.shape

    table_rm = _tc_repack(table.T).reshape(VOCAB_ROWS, EMBED_DIM)
    x2 = _sc_gather(sents.T.astype(jnp.int32), table_rm)
    out2d = _tc_out_transpose(x2, seq_len, batch)
    return out2d.reshape(seq_len, EMBED_DIM, batch).transpose(2, 0, 1)
